# Initial kernel scaffold; baseline (speedup 1.0000x reference)
import jax
import jax.numpy as jnp
from jax.experimental import pallas as pl


def kernel(coords, features, t_embed, queries):
    raise NotImplementedError("write your pallas kernel here")


# speedup vs baseline: 19.3992x; 19.3992x over previous
import functools

import jax
import jax.numpy as jnp
from jax import lax
from jax.experimental import pallas as pl
from jax.experimental.pallas import tpu as pltpu
from jax.experimental.pallas import tpu_sc as plsc

_RADIUS2 = 0.1 * 0.1
_K = 32


_MT = 256
_NT = 512


def _ballq_kernel(q_ref, c_ref, lt_ref, out_ref, acc_s, cm_s,
                  *, n_points, k, mt, nt):
    b = pl.program_id(0)
    i = pl.program_id(2)
    n_chunks = n_points // nt

    @pl.when(i == 0)
    def _():
        acc_s[...] = jnp.zeros((mt, k), jnp.float32)
        cm_s[...] = jnp.zeros((mt, 1), jnp.float32)

    qx = q_ref[0, :, 0:1]
    qy = q_ref[0, :, 1:2]
    qz = q_ref[0, :, 2:3]
    cx = c_ref[0, 0:1, :]
    cy = c_ref[0, 1:2, :]
    cz = c_ref[0, 2:3, :]
    dx = qx - cx
    dy = qy - cy
    dz = qz - cz
    d2 = dx * dx + dy * dy + dz * dz
    m = jnp.where(d2 <= _RADIUS2, 1.0, 0.0).astype(jnp.float32)
    lc = jnp.dot(m, lt_ref[...], preferred_element_type=jnp.float32)
    cnt = cm_s[...] + lc
    cols = [
        jnp.sum(jnp.where(cnt <= jnp.float32(kk), 1.0, 0.0),
                axis=1, keepdims=True)
        for kk in range(k)
    ]
    acc_s[...] += jnp.concatenate(cols, axis=1)
    cm_s[...] += lc[:, nt - 1:nt]

    @pl.when(i == n_chunks - 1)
    def _():
        acc = acc_s[...]
        nf = jnp.float32(n_points)
        first = acc[:, 0:1]
        first = jnp.where(first >= nf, 0.0, first)
        idx = jnp.where(acc >= nf, first, acc)
        out_ref[0, :, :] = idx.astype(jnp.int32) + b * n_points


def _ball_query(queries, coords_t):
    B, M, _ = queries.shape
    N = coords_t.shape[2]
    ra = lax.broadcasted_iota(jnp.int32, (_NT, _NT), 0)
    rb = lax.broadcasted_iota(jnp.int32, (_NT, _NT), 1)
    lt = jnp.where(ra <= rb, 1.0, 0.0).astype(jnp.float32)
    grid = (B, M // _MT, N // _NT)
    return pl.pallas_call(
        functools.partial(_ballq_kernel, n_points=N, k=_K, mt=_MT, nt=_NT),
        grid=grid,
        in_specs=[
            pl.BlockSpec((1, _MT, 3), lambda b, m, i: (b, m, 0)),
            pl.BlockSpec((1, 3, _NT), lambda b, m, i: (b, 0, i)),
            pl.BlockSpec((_NT, _NT), lambda b, m, i: (0, 0)),
        ],
        out_specs=pl.BlockSpec((1, _MT, _K), lambda b, m, i: (b, m, 0)),
        out_shape=jax.ShapeDtypeStruct((B, M, _K), jnp.int32),
        scratch_shapes=[
            pltpu.VMEM((_MT, _K), jnp.float32),
            pltpu.VMEM((_MT, 1), jnp.float32),
        ],
    )(queries, coords_t, lt)


_D = 128
_QW = 16
_RC = 128


def _sc_gather(table, gidx, qrep):
    rows = gidx.shape[0]
    info = plsc.get_sparse_core_info()
    nc, ns = info.num_cores, info.num_subcores
    nw = nc * ns
    per_w = rows // nw
    n_chunks = per_w // _RC
    mesh = plsc.VectorSubcoreMesh(core_axis_name="c", subcore_axis_name="s")

    @functools.partial(
        pl.kernel,
        mesh=mesh,
        out_type=jax.ShapeDtypeStruct((rows, _D), jnp.float32),
        scratch_types=[
            pltpu.VMEM((_RC,), jnp.int32),
            pltpu.VMEM((_RC, _D), jnp.float32),
            pltpu.VMEM((_RC, _QW), jnp.float32),
            pltpu.SemaphoreType.DMA,
        ],
    )
    def k(table_hbm, gidx_hbm, qrep_hbm, out_hbm, idx_v, rows_v, q_v, sem):
        wid = lax.axis_index("s") * nc + lax.axis_index("c")
        base = wid * per_w

        def chunk(t, _):
            gbase = base + t * _RC
            pltpu.sync_copy(gidx_hbm.at[pl.ds(gbase, _RC)], idx_v)
            pltpu.async_copy(table_hbm.at[idx_v], rows_v, sem).wait()
            pltpu.sync_copy(qrep_hbm.at[pl.ds(gbase, _RC)], q_v)

            def sub(r, _):
                rows_v[r, 0:_QW] = rows_v[r, 0:_QW] - q_v[r, :]
                return ()

            lax.fori_loop(0, _RC, sub, ())
            pltpu.sync_copy(rows_v, out_hbm.at[pl.ds(gbase, _RC)])
            return ()

        lax.fori_loop(0, n_chunks, chunk, ())

    return k(table, gidx, qrep)


def kernel(coords, features, t_embed, queries):
    B, N, _ = coords.shape
    M = queries.shape[1]
    C = features.shape[1]
    Ct = t_embed.shape[1]

    coords_t = jnp.transpose(coords, (0, 2, 1))
    gidx = _ball_query(queries, coords_t).reshape(B * M * _K)

    pad = _D - 3 - C - Ct
    table = jnp.concatenate(
        [coords,
         jnp.transpose(features, (0, 2, 1)),
         jnp.transpose(t_embed, (0, 2, 1)),
         jnp.zeros((B, N, pad), jnp.float32)],
        axis=-1).reshape(B * N, _D)

    qpad = jnp.concatenate(
        [queries, jnp.zeros((B, M, _QW - 3), jnp.float32)], axis=-1)
    qrep = jnp.broadcast_to(qpad[:, :, None, :],
                            (B, M, _K, _QW)).reshape(B * M * _K, _QW)

    g = _sc_gather(table, gidx, qrep).reshape(B, M, _K, _D)
    grouped_features = jnp.transpose(g[..., 0:3 + C], (0, 3, 1, 2))
    gt = jnp.transpose(g[..., 3 + C:3 + C + Ct], (0, 3, 1, 2))
    return (grouped_features, gt)
